# Initial kernel scaffold; baseline (speedup 1.0000x reference)
import jax
import jax.numpy as jnp
from jax.experimental import pallas as pl


def kernel(x_enc, x_mark_enc, x_dec, x_mark_dec, conv_W, temp_W, Wqk, Wv, Wo, bo, c1W, c1b, c2W, c2b, n1g, n1b, n2g, n2b, fng, fnb, pW, pb, fW, fb):
    raise NotImplementedError("write your pallas kernel here")


# speedup vs baseline: 4.8344x; 4.8344x over previous
import functools
import math

import jax
import jax.numpy as jnp
from jax import lax
from jax.experimental import pallas as pl
from jax.experimental.pallas import tpu as pltpu
from jax.experimental.pallas import tpu_sc as plsc

_B = 2
_SEQ = 1536
_PRED = 512
_ENC_IN = 7
_DMARK = 4
_DM = 1024
_NH = 16
_DFF = 2048
_L = 2
_BUCKET = 64
_NHASH = 4
_CO = 7
_NCLS = 10
_T = _SEQ + _PRED
_DH = _DM // _NH
_BH = _B * _NH
_NB = _T // _BUCKET
_P = _BH * _NHASH
_NCG = _NHASH * _NB
_PW = 2 * _DH


def _embed_body(xin_ref, w_ref, pe_ref, o_ref):
    o_ref[...] = (
        jnp.dot(xin_ref[...], w_ref[...], preferred_element_type=jnp.float32)
        + pe_ref[...]
    )


def _embed(xcat, w_emb, pe):
    tb = 512
    return pl.pallas_call(
        _embed_body,
        grid=(_B * _T // tb,),
        in_specs=[
            pl.BlockSpec((tb, 32), lambda i: (i, 0)),
            pl.BlockSpec((32, _DM), lambda i: (0, 0)),
            pl.BlockSpec((tb, _DM), lambda i: (i % (_T // tb), 0)),
        ],
        out_specs=pl.BlockSpec((tb, _DM), lambda i: (i, 0)),
        out_shape=jax.ShapeDtypeStruct((_B * _T, _DM), jnp.float32),
    )(xcat, w_emb, pe)


def _proj_body(x_ref, wqk_ref, wv_ref, qv_ref):
    x = x_ref[0]
    qk = jnp.dot(x, wqk_ref[0].T, preferred_element_type=jnp.float32)
    v = jnp.dot(x, wv_ref[0].T, preferred_element_type=jnp.float32)
    qv_ref[0] = jnp.concatenate([qk, v], axis=1)


def _proj(x, wqk, wv):
    tb = 512
    grid = (_B, _NH, _T // tb)
    return pl.pallas_call(
        _proj_body,
        grid=grid,
        in_specs=[
            pl.BlockSpec((1, tb, _DM), lambda b, h, i: (b, i, 0)),
            pl.BlockSpec((1, _DH, _DM), lambda b, h, i: (h, 0, 0)),
            pl.BlockSpec((1, _DH, _DM), lambda b, h, i: (h, 0, 0)),
        ],
        out_specs=pl.BlockSpec((1, tb, _PW),
                               lambda b, h, i: (b * _NH + h, i, 0)),
        out_shape=jax.ShapeDtypeStruct((_BH, _T, _PW), jnp.float32),
    )(x, wqk.reshape(_NH, _DH, _DM), wv.reshape(_NH, _DH, _DM))


def _dest_body(qv_ref, rot_ref, tri_ref, dest_ref):
    bh = pl.program_id(0)
    rT = jax.lax.dot_general(rot_ref[...], qv_ref[0, :, :_DH],
                             (((0,), (1,)), ((), ())),
                             preferred_element_type=jnp.float32)
    nh2 = _NB // 2
    for h in range(_NHASH):
        rh = rT[h * nh2:(h + 1) * nh2, :]
        s = jnp.concatenate([rh, -rh], axis=0)
        m = jnp.max(s, axis=0, keepdims=True)
        si = jax.lax.broadcasted_iota(
            jnp.int32, (_NB, _T), 0).astype(jnp.float32)
        bucket = jnp.min(jnp.where(s >= m, si, float(_NB)), axis=0,
                         keepdims=True)
        oh = jnp.where(si == bucket, 1.0, 0.0)
        c = oh
        k = 1
        while k < _T:
            c = c + jnp.concatenate(
                [jnp.zeros((_NB, k), jnp.float32), c[:, :-k]], axis=1)
            k *= 2
        rank = jnp.sum(oh * (c - oh), axis=0, keepdims=True)
        counts = c[:, _T - 1:_T]
        offs = jnp.dot(tri_ref[...], counts,
                       preferred_element_type=jnp.float32)
        base = jnp.sum(oh * offs, axis=0, keepdims=True)
        dest = (base + rank).astype(jnp.int32) + ((bh * _NHASH + h) * _T)
        dest_ref[0, h:h + 1, :] = dest


def _dest(qv, rot64):
    tri = (jax.lax.broadcasted_iota(jnp.float32, (_NB, _NB), 1)
           < jax.lax.broadcasted_iota(jnp.float32, (_NB, _NB), 0)
           ).astype(jnp.float32)
    return pl.pallas_call(
        _dest_body,
        grid=(_BH,),
        in_specs=[
            pl.BlockSpec((1, _T, _PW), lambda b: (b, 0, 0)),
            pl.BlockSpec((_DH, _NHASH * _NB // 2), lambda b: (0, 0)),
            pl.BlockSpec((_NB, _NB), lambda b: (0, 0)),
        ],
        out_specs=pl.BlockSpec((1, _NHASH, _T), lambda b: (b, 0, 0)),
        out_shape=jax.ShapeDtypeStruct((_BH, _NHASH, _T), jnp.int32),
    )(qv, rot64, tri)


def _bmask_body(dc_ref, dp_ref, out_ref):
    b = pl.program_id(0)
    h = pl.program_id(1)
    hp = jax.lax.rem(h + _NHASH - 1, _NHASH)
    pc = b * _NHASH + h
    pp = b * _NHASH + hp
    lc = dc_ref[0] - pc * _T
    lp = dp_ref[0] - pp * _T
    ri = jax.lax.broadcasted_iota(jnp.int32, (_BUCKET, _T), 0)
    uc = (lc == ri).astype(jnp.float32)
    up = (lp == (ri + (_T - _BUCKET))).astype(jnp.float32)
    out_ref[0, 0] = jax.lax.dot_general(
        uc, up, (((1,), (1,)), ((), ())),
        preferred_element_type=jnp.float32)


def _bmask(dest_g):
    grid = (_BH, _NHASH)
    return pl.pallas_call(
        _bmask_body,
        grid=grid,
        in_specs=[
            pl.BlockSpec((1, 1, _T), lambda b, h: (b * _NHASH + h, 0, 0)),
            pl.BlockSpec((1, 1, _T),
                         lambda b, h: (b * _NHASH + (h + _NHASH - 1) % _NHASH,
                                       0, 0)),
        ],
        out_specs=pl.BlockSpec((1, 1, _BUCKET, _BUCKET),
                               lambda b, h: (b, h, 0, 0)),
        out_shape=jax.ShapeDtypeStruct((_BH, _NHASH, _BUCKET, _BUCKET),
                                       jnp.float32),
    )(dest_g.reshape(_P, 1, _T), dest_g.reshape(_P, 1, _T))


_CK = 128


def _sc_scatter_body(qv_hbm, dest_hbm, sqv_hbm, qv_v, idx_v, sem0):
    nc = 2
    bh = lax.axis_index("s") * nc + lax.axis_index("c")

    def chunk(i, carry):
        pltpu.sync_copy(qv_hbm.at[pl.ds(bh * _T + i * _CK, _CK), :], qv_v)

        def per_h(h, c2):
            pltpu.sync_copy(
                dest_hbm.at[bh * _NHASH + h, pl.ds(i * _CK, _CK)], idx_v)
            pltpu.async_copy(qv_v, sqv_hbm.at[idx_v], sem0).wait()
            return c2

        return lax.fori_loop(0, _NHASH, per_h, carry)

    lax.fori_loop(0, _T // _CK, chunk, 0)


def _sc_scatter(qv_flat, dest_g):
    mesh = plsc.VectorSubcoreMesh(core_axis_name="c", subcore_axis_name="s")
    f = pl.kernel(
        _sc_scatter_body,
        out_type=jax.ShapeDtypeStruct((_P * _T, _PW), jnp.float32),
        mesh=mesh,
        scratch_types=[
            pltpu.VMEM((_CK, _PW), jnp.float32),
            pltpu.VMEM((_CK,), jnp.int32),
            pltpu.SemaphoreType.DMA,
        ],
    )
    return f(qv_flat, dest_g)


def _sc_gather_body(sol_hbm, dest_hbm, og_hbm, buf_v, idx_v, sem0):
    nc = 2
    bh = lax.axis_index("s") * nc + lax.axis_index("c")

    def chunk(i, carry):
        def per_h(h, c2):
            p = bh * _NHASH + h
            pltpu.sync_copy(dest_hbm.at[p, pl.ds(i * _CK, _CK)], idx_v)
            pltpu.async_copy(sol_hbm.at[idx_v], buf_v, sem0).wait()
            pltpu.sync_copy(
                buf_v, og_hbm.at[pl.ds(p * _T + i * _CK, _CK), :])
            return c2

        return lax.fori_loop(0, _NHASH, per_h, carry)

    lax.fori_loop(0, _T // _CK, chunk, 0)


def _sc_gather(sol, dest_g):
    mesh = plsc.VectorSubcoreMesh(core_axis_name="c", subcore_axis_name="s")
    f = pl.kernel(
        _sc_gather_body,
        out_type=jax.ShapeDtypeStruct((_P * _T, _PW), jnp.float32),
        mesh=mesh,
        scratch_types=[
            pltpu.VMEM((_CK, _PW), jnp.float32),
            pltpu.VMEM((_CK,), jnp.int32),
            pltpu.SemaphoreType.DMA,
        ],
    )
    return f(sol, dest_g)


_ACH = 8


def _attn_body(qc_ref, qp_ref, bm_ref, sol_ref):
    scale = _DH ** -0.5
    i = pl.program_id(1)
    at_boundary = (jax.lax.rem(i, _NCG // _ACH // _NHASH) == 0
                   ).astype(jnp.float32)
    ri = jax.lax.broadcasted_iota(jnp.int32, (_BUCKET, 2 * _BUCKET), 0)
    ci = jax.lax.broadcasted_iota(jnp.int32, (_BUCKET, 2 * _BUCKET), 1)
    diag = (ci == ri + _BUCKET).astype(jnp.float32)
    for j in range(_ACH):
        cur = qc_ref[0, j]
        prev = qp_ref[0, 0] if j == 0 else qc_ref[0, j - 1]
        q = cur[:, :_DH]
        kfull = jnp.concatenate([prev[:, :_DH], q], axis=0)
        vfull = jnp.concatenate([prev[:, _DH:], cur[:, _DH:]], axis=0)
        nrm = jnp.sqrt(jnp.sum(kfull * kfull, axis=1, keepdims=True))
        kn = kfull / (nrm + 1e-12)
        dots = jax.lax.dot_general(
            q, kn, (((1,), (1,)), ((), ())),
            preferred_element_type=jnp.float32) * scale
        mask = diag
        if j == 0:
            bmask = jnp.concatenate(
                [bm_ref[0, 0] * at_boundary,
                 jnp.zeros((_BUCKET, _BUCKET), jnp.float32)], axis=1)
            mask = jnp.maximum(diag, bmask)
        dots = jnp.where(mask > 0.5, -5e4, dots)
        m = jnp.max(dots, axis=1, keepdims=True)
        e = jnp.exp(dots - m)
        s = jnp.sum(e, axis=1, keepdims=True)
        lse = m + jnp.log(s)
        o = jnp.dot(e / s, vfull,
                    preferred_element_type=jnp.float32)
        sol_ref[0, j] = jnp.concatenate(
            [o, jnp.broadcast_to(lse, (_BUCKET, _DH))], axis=1)


def _attention(sqv, bmask):
    grid = (_BH, _NCG // _ACH)
    prev = lambda b, i: (b, (i * _ACH - 1) % _NCG, 0, 0)
    return pl.pallas_call(
        _attn_body,
        grid=grid,
        in_specs=[
            pl.BlockSpec((1, _ACH, _BUCKET, _PW), lambda b, i: (b, i, 0, 0)),
            pl.BlockSpec((1, 1, _BUCKET, _PW), prev),
            pl.BlockSpec((1, 1, _BUCKET, _BUCKET),
                         lambda b, i: (b, i // (_NCG // _ACH // _NHASH),
                                       0, 0)),
        ],
        out_specs=pl.BlockSpec((1, _ACH, _BUCKET, _PW),
                               lambda b, i: (b, i, 0, 0)),
        out_shape=jax.ShapeDtypeStruct((_BH, _NCG, _BUCKET, _PW),
                                       jnp.float32),
    )(sqv, sqv, bmask)


def _combine_body(og_ref, out_ref):
    tb = og_ref.shape[2]
    halves = []
    for k in range(2):
        ls = [og_ref[k, h, :, _DH:_DH + 1] for h in range(_NHASH)]
        mx = ls[0]
        for h in range(1, _NHASH):
            mx = jnp.maximum(mx, ls[h])
        es = [jnp.exp(l - mx) for l in ls]
        tot = es[0]
        for h in range(1, _NHASH):
            tot = tot + es[h]
        acc = jnp.zeros((tb, _DH), jnp.float32)
        for h in range(_NHASH):
            acc = acc + og_ref[k, h, :, :_DH] * (es[h] / tot)
        halves.append(acc)
    out_ref[0] = jnp.concatenate(halves, axis=1)


def _combine(og):
    tb = 512
    grid = (_B, _NH // 2, _T // tb)
    return pl.pallas_call(
        _combine_body,
        grid=grid,
        in_specs=[
            pl.BlockSpec((2, _NHASH, tb, _PW),
                         lambda b, j, i: (b * (_NH // 2) + j, 0, i, 0)),
        ],
        out_specs=pl.BlockSpec((1, tb, 2 * _DH), lambda b, j, i: (b, i, j)),
        out_shape=jax.ShapeDtypeStruct((_B, _T, _DM), jnp.float32),
    )(og)


def _ln(x, g, b, eps=1e-5):
    mu = jnp.mean(x, axis=-1, keepdims=True)
    var = jnp.mean((x - mu) ** 2, axis=-1, keepdims=True)
    return (x - mu) / jnp.sqrt(var + eps) * g + b


def _wo_body(a_ref, w_ref, bo_ref, res_ref, g_ref, b_ref, o_ref):
    xo = (jax.lax.dot_general(
        a_ref[...], w_ref[...], (((1,), (1,)), ((), ())),
        preferred_element_type=jnp.float32) + bo_ref[...] + res_ref[...])
    o_ref[...] = _ln(xo, g_ref[...], b_ref[...])


def _wo_ln(attn2, wo, bo, res2, g, b):
    tb = 256
    return pl.pallas_call(
        _wo_body,
        grid=(_B * _T // tb,),
        in_specs=[
            pl.BlockSpec((tb, _DM), lambda i: (i, 0)),
            pl.BlockSpec((_DM, _DM), lambda i: (0, 0)),
            pl.BlockSpec((1, _DM), lambda i: (0, 0)),
            pl.BlockSpec((tb, _DM), lambda i: (i, 0)),
            pl.BlockSpec((1, _DM), lambda i: (0, 0)),
            pl.BlockSpec((1, _DM), lambda i: (0, 0)),
        ],
        out_specs=pl.BlockSpec((tb, _DM), lambda i: (i, 0)),
        out_shape=jax.ShapeDtypeStruct((_B * _T, _DM), jnp.float32),
    )(attn2, wo, bo.reshape(1, _DM), res2, g.reshape(1, _DM),
      b.reshape(1, _DM))


def _ffn1_body(y_ref, w_ref, b_ref, o_ref):
    h = (jax.lax.dot_general(
        y_ref[...], w_ref[...], (((1,), (1,)), ((), ())),
        preferred_element_type=jnp.float32) + b_ref[...])
    o_ref[...] = 0.5 * h * (1.0 + jax.lax.erf(h * (2.0 ** -0.5)))


def _ffn1(y2, c1w, c1b):
    tb = 256
    return pl.pallas_call(
        _ffn1_body,
        grid=(_B * _T // tb,),
        in_specs=[
            pl.BlockSpec((tb, _DM), lambda i: (i, 0)),
            pl.BlockSpec((_DFF, _DM), lambda i: (0, 0)),
            pl.BlockSpec((1, _DFF), lambda i: (0, 0)),
        ],
        out_specs=pl.BlockSpec((tb, _DFF), lambda i: (i, 0)),
        out_shape=jax.ShapeDtypeStruct((_B * _T, _DFF), jnp.float32),
    )(y2, c1w, c1b.reshape(1, _DFF))


def _ffn2_body(h_ref, w_ref, b_ref, res_ref, g_ref, bb_ref, o_ref):
    z = (jax.lax.dot_general(
        h_ref[...], w_ref[...], (((1,), (1,)), ((), ())),
        preferred_element_type=jnp.float32) + b_ref[...] + res_ref[...])
    o_ref[...] = _ln(z, g_ref[...], bb_ref[...])


def _ffn2(hh, c2w, c2b, res2, g, b):
    tb = 256
    return pl.pallas_call(
        _ffn2_body,
        grid=(_B * _T // tb,),
        in_specs=[
            pl.BlockSpec((tb, _DFF), lambda i: (i, 0)),
            pl.BlockSpec((_DM, _DFF), lambda i: (0, 0)),
            pl.BlockSpec((1, _DM), lambda i: (0, 0)),
            pl.BlockSpec((tb, _DM), lambda i: (i, 0)),
            pl.BlockSpec((1, _DM), lambda i: (0, 0)),
            pl.BlockSpec((1, _DM), lambda i: (0, 0)),
        ],
        out_specs=pl.BlockSpec((tb, _DM), lambda i: (i, 0)),
        out_shape=jax.ShapeDtypeStruct((_B * _T, _DM), jnp.float32),
    )(hh, c2w, c2b.reshape(1, _DM), res2, g.reshape(1, _DM),
      b.reshape(1, _DM))


def _head_body(x_ref, g_ref, b_ref, pw_ref, pb_ref, o_ref):
    y = _ln(x_ref[...], g_ref[...], b_ref[...])
    o_ref[...] = (jax.lax.dot_general(
        y, pw_ref[...], (((1,), (1,)), ((), ())),
        preferred_element_type=jnp.float32) + pb_ref[...])


def _head(x2, fng, fnb, pw, pb):
    tb = 256
    return pl.pallas_call(
        _head_body,
        grid=(_B * _T // tb,),
        in_specs=[
            pl.BlockSpec((tb, _DM), lambda i: (i, 0)),
            pl.BlockSpec((1, _DM), lambda i: (0, 0)),
            pl.BlockSpec((1, _DM), lambda i: (0, 0)),
            pl.BlockSpec((_CO, _DM), lambda i: (0, 0)),
            pl.BlockSpec((1, _CO), lambda i: (0, 0)),
        ],
        out_specs=pl.BlockSpec((tb, _CO), lambda i: (i, 0)),
        out_shape=jax.ShapeDtypeStruct((_B * _T, _CO), jnp.float32),
    )(x2, fng.reshape(1, _DM), fnb.reshape(1, _DM), pw, pb.reshape(1, _CO))


def _cls_body(x_ref, w_ref, b_ref, o_ref):
    z = (jax.lax.dot_general(
        x_ref[...], w_ref[...], (((1,), (1,)), ((), ())),
        preferred_element_type=jnp.float32) + b_ref[...])
    m = jnp.max(z, axis=1, keepdims=True)
    e = jnp.exp(z - m)
    o_ref[...] = e / jnp.sum(e, axis=1, keepdims=True)


def _cls(xf, fw, fb):
    return pl.pallas_call(
        _cls_body,
        grid=(1,),
        in_specs=[
            pl.BlockSpec((8, _CO * _PRED), lambda i: (0, 0)),
            pl.BlockSpec((_NCLS, _CO * _PRED), lambda i: (0, 0)),
            pl.BlockSpec((1, _NCLS), lambda i: (0, 0)),
        ],
        out_specs=pl.BlockSpec((8, _NCLS), lambda i: (0, 0)),
        out_shape=jax.ShapeDtypeStruct((8, _NCLS), jnp.float32),
    )(xf, fw, fb.reshape(1, _NCLS))


def _pos_embedding():
    position = jnp.arange(_T, dtype=jnp.float32)[:, None]
    div = jnp.exp(jnp.arange(0, _DM, 2, dtype=jnp.float32)
                  * -(math.log(10000.0) / _DM))
    pe = jnp.zeros((_T, _DM), jnp.float32)
    pe = pe.at[:, 0::2].set(jnp.sin(position * div))
    pe = pe.at[:, 1::2].set(jnp.cos(position * div))
    return pe


def kernel(x_enc, x_mark_enc, x_dec, x_mark_dec, conv_W, temp_W, Wqk, Wv,
           Wo, bo, c1W, c1b, c2W, c2b, n1g, n1b, n2g, n2b, fng, fnb, pW,
           pb, fW, fb):
    xe = jnp.concatenate([x_enc, x_dec[:, -_PRED:, :]], axis=1)
    xm = jnp.concatenate([x_mark_enc, x_mark_dec[:, -_PRED:, :]], axis=1)

    xp = jnp.concatenate([xe[:, -1:], xe, xe[:, :1]], axis=1)
    xcat = jnp.concatenate(
        [xp[:, :-2], xp[:, 1:-1], xp[:, 2:], xm], axis=-1)
    xcat = jnp.pad(xcat, ((0, 0), (0, 0), (0, 7)))
    w_emb = jnp.concatenate(
        [conv_W[:, :, 0].T, conv_W[:, :, 1].T, conv_W[:, :, 2].T, temp_W.T,
         jnp.zeros((7, _DM), jnp.float32)], axis=0)
    x2 = _embed(xcat.reshape(_B * _T, 32), w_emb, _pos_embedding())
    x = x2.reshape(_B, _T, _DM)

    for l in range(_L):
        rot = jax.random.normal(
            jax.random.fold_in(jax.random.key(42), l),
            (_DH, _NHASH, _NB // 2), dtype=jnp.float32)
        qv = _proj(x, Wqk[l], Wv[l])
        dest_g = _dest(qv, rot.reshape(_DH, _NHASH * _NB // 2))
        dest_g = dest_g.reshape(_P, _T)
        bmask = _bmask(dest_g)
        sqv = _sc_scatter(qv.reshape(_BH * _T, _PW), dest_g)
        sol = _attention(sqv.reshape(_BH, _NCG, _BUCKET, _PW), bmask)
        og = _sc_gather(sol.reshape(_P * _T, _PW), dest_g)
        attn = _combine(og.reshape(_BH, _NHASH, _T, _PW))
        x2 = _wo_ln(attn.reshape(_B * _T, _DM), Wo[l], bo[l],
                    x.reshape(_B * _T, _DM), n1g[l], n1b[l])
        hh = _ffn1(x2, c1W[l], c1b[l])
        x2 = _ffn2(hh, c2W[l], c2b[l], x2, n2g[l], n2b[l])
        x = x2.reshape(_B, _T, _DM)

    outp = _head(x.reshape(_B * _T, _DM), fng, fnb, pW, pb)
    outp = outp.reshape(_B, _T, _CO)[:, -_PRED:, :].reshape(_B, _PRED * _CO)
    outp = jnp.pad(outp, ((0, 8 - _B), (0, 0)))
    return _cls(outp, fW, fb)[:_B]
